# baseline (speedup 1.0000x reference)
import functools

import jax
import jax.numpy as jnp
from jax import lax
from jax.experimental import pallas as pl
from jax.experimental.pallas import tpu as pltpu
from jax.experimental.pallas import tpu_sc as plsc

HIDDEN = 128
NUM_CORES = 2
NUM_SUBCORES = 16
NW = NUM_CORES * NUM_SUBCORES
HALF = 4

_mesh = plsc.VectorSubcoreMesh(core_axis_name="c", subcore_axis_name="s")


@functools.partial(jax.jit, static_argnames=("chunk", "nchunk"))
def _sc_gather(idx, table, *, chunk, nchunk):
    b = idx.shape[0]
    bpw = b // NW
    half_rows = HALF * chunk
    nhalf = bpw // half_rows
    npairs = nhalf // 2

    @functools.partial(
        pl.kernel,
        mesh=_mesh,
        out_type=jax.ShapeDtypeStruct((b, HIDDEN), jnp.float32),
        scratch_types=[
            pltpu.VMEM((bpw,), jnp.int32),
            pltpu.VMEM((2, half_rows, HIDDEN), jnp.float32),
            pltpu.VMEM_SHARED((1000, HIDDEN), jnp.float32),
            [pltpu.SemaphoreType.DMA] * 2,
            [pltpu.SemaphoreType.DMA] * 2,
        ],
    )
    def body(idx_hbm, table_hbm, out_hbm, idx_v, rows_v, table_sh, gsems, ssems):
        sid = lax.axis_index("s")
        wid = sid * NUM_CORES + lax.axis_index("c")
        base = wid * bpw

        @pl.when(sid == 0)
        def _stage_table():
            pltpu.sync_copy(table_hbm, table_sh)

        pltpu.sync_copy(idx_hbm.at[pl.ds(base, bpw)], idx_v)
        plsc.subcore_barrier()

        def gather_half(m, h):
            for j in range(HALF):
                src = table_sh.at[
                    idx_v.at[pl.ds(m * half_rows + j * chunk, chunk)]
                ]
                pltpu.async_copy(
                    src, rows_v.at[h, pl.ds(j * chunk, chunk)], gsems[h]
                )

        def gather_wait(h):
            for j in range(HALF):
                pltpu.make_async_copy(
                    table_hbm.at[pl.ds(0, chunk)],
                    rows_v.at[h, pl.ds(0, chunk)],
                    gsems[h],
                ).wait()

        def scatter_half(m, h):
            dst = out_hbm.at[pl.ds(base + m * half_rows, half_rows)]
            pltpu.async_copy(rows_v.at[h], dst, ssems[h])

        def scatter_wait(h):
            pltpu.make_async_copy(
                rows_v.at[h],
                out_hbm.at[pl.ds(base, half_rows)],
                ssems[h],
            ).wait()

        gather_half(0, 0)
        gather_half(1, 1)

        def pair(p, _):
            for h in range(2):
                gather_wait(h)
                scatter_half(2 * p + h, h)

                @pl.when(2 * p + h + 2 < nhalf)
                def _refill():
                    scatter_wait(h)
                    gather_half(2 * p + h + 2, h)

            return ()

        lax.fori_loop(0, npairs, pair, ())
        scatter_wait(0)
        scatter_wait(1)

    return body(idx, table)


def kernel(input_ids, table):
    ids_flat = input_ids.reshape(-1).astype(jnp.int32)
    b = ids_flat.shape[0]
    chunk = 80
    nchunk = (b // NW) // chunk
    assert b % (NW * chunk * HALF * 2) == 0
    out = _sc_gather(ids_flat, table, chunk=chunk, nchunk=nchunk)
    return out.reshape(input_ids.shape + (HIDDEN,))


# speedup vs baseline: 1.0021x; 1.0021x over previous
import functools

import jax
import jax.numpy as jnp
from jax import lax
from jax.experimental import pallas as pl
from jax.experimental.pallas import tpu as pltpu
from jax.experimental.pallas import tpu_sc as plsc

HIDDEN = 128
NUM_CORES = 2
NUM_SUBCORES = 16
NW = NUM_CORES * NUM_SUBCORES
NBUF = 5

_mesh = plsc.VectorSubcoreMesh(core_axis_name="c", subcore_axis_name="s")


@functools.partial(jax.jit, static_argnames=("chunk", "nchunk"))
def _sc_gather(idx, table, *, chunk, nchunk):
    b = idx.shape[0]
    bpw = b // NW
    ngroups = nchunk // NBUF

    @functools.partial(
        pl.kernel,
        mesh=_mesh,
        out_type=jax.ShapeDtypeStruct((b, HIDDEN), jnp.float32),
        scratch_types=[
            pltpu.VMEM((bpw,), jnp.int32),
            pltpu.VMEM((NBUF, chunk, HIDDEN), jnp.float32),
            pltpu.VMEM_SHARED((1000, HIDDEN), jnp.float32),
            [pltpu.SemaphoreType.DMA] * NBUF,
            [pltpu.SemaphoreType.DMA] * NBUF,
        ],
    )
    def body(idx_hbm, table_hbm, out_hbm, idx_v, rows_v, table_sh, gsems, ssems):
        sid = lax.axis_index("s")
        wid = sid * NUM_CORES + lax.axis_index("c")
        base = wid * bpw

        @pl.when(sid == 0)
        def _stage_table():
            pltpu.sync_copy(table_hbm, table_sh)

        pltpu.sync_copy(idx_hbm.at[pl.ds(base, bpw)], idx_v)
        plsc.subcore_barrier()

        def gather(g, buf):
            src = table_sh.at[idx_v.at[pl.ds(g * chunk, chunk)]]
            pltpu.async_copy(src, rows_v.at[buf], gsems[buf])

        def gather_wait(buf):
            pltpu.make_async_copy(
                table_hbm.at[pl.ds(0, chunk)], rows_v.at[buf], gsems[buf]
            ).wait()

        def scatter(g, buf):
            dst = out_hbm.at[pl.ds(base + g * chunk, chunk)]
            pltpu.async_copy(rows_v.at[buf], dst, ssems[buf])

        def scatter_wait(buf):
            pltpu.make_async_copy(
                rows_v.at[buf], out_hbm.at[pl.ds(base, chunk)], ssems[buf]
            ).wait()

        for buf in range(NBUF):
            gather(buf, buf)

        def group(p, _):
            for buf in range(NBUF):
                gather_wait(buf)
                scatter(p * NBUF + buf, buf)

            @pl.when(p + 1 < ngroups)
            def _refill():
                for buf in range(NBUF):
                    scatter_wait(buf)
                    gather((p + 1) * NBUF + buf, buf)

            return ()

        lax.fori_loop(0, ngroups, group, ())
        for buf in range(NBUF):
            scatter_wait(buf)

    return body(idx, table)


def kernel(input_ids, table):
    ids_flat = input_ids.reshape(-1).astype(jnp.int32)
    b = ids_flat.shape[0]
    chunk = 128
    nchunk = (b // NW) // chunk
    assert b % (NW * chunk * NBUF) == 0
    out = _sc_gather(ids_flat, table, chunk=chunk, nchunk=nchunk)
    return out.reshape(input_ids.shape + (HIDDEN,))


# speedup vs baseline: 1.0196x; 1.0174x over previous
import functools

import jax
import jax.numpy as jnp
from jax import lax
from jax.experimental import pallas as pl
from jax.experimental.pallas import tpu as pltpu
from jax.experimental.pallas import tpu_sc as plsc

HIDDEN = 128
NUM_CORES = 2
NUM_SUBCORES = 16
NW = NUM_CORES * NUM_SUBCORES
NBUF = 8

_mesh = plsc.VectorSubcoreMesh(core_axis_name="c", subcore_axis_name="s")


@functools.partial(jax.jit, static_argnames=("chunk", "nchunk"))
def _sc_gather(idx, table, *, chunk, nchunk):
    b = idx.shape[0]
    bpw = b // NW
    ngroups = nchunk // NBUF

    @functools.partial(
        pl.kernel,
        mesh=_mesh,
        out_type=jax.ShapeDtypeStruct((b, HIDDEN), jnp.float32),
        scratch_types=[
            pltpu.VMEM((bpw,), jnp.int32),
            pltpu.VMEM((NBUF, chunk, HIDDEN), jnp.float32),
            pltpu.VMEM_SHARED((1000, HIDDEN), jnp.float32),
            [pltpu.SemaphoreType.DMA] * NBUF,
            [pltpu.SemaphoreType.DMA] * NBUF,
        ],
    )
    def body(idx_hbm, table_hbm, out_hbm, idx_v, rows_v, table_sh, gsems, ssems):
        sid = lax.axis_index("s")
        wid = sid * NUM_CORES + lax.axis_index("c")
        base = wid * bpw

        @pl.when(sid == 0)
        def _stage_table():
            pltpu.sync_copy(table_hbm, table_sh)

        pltpu.sync_copy(idx_hbm.at[pl.ds(base, bpw)], idx_v)
        plsc.subcore_barrier()

        def gather(g, buf):
            src = table_sh.at[idx_v.at[pl.ds(g * chunk, chunk)]]
            pltpu.async_copy(src, rows_v.at[buf], gsems[buf])

        def gather_wait(buf):
            pltpu.make_async_copy(
                table_hbm.at[pl.ds(0, chunk)], rows_v.at[buf], gsems[buf]
            ).wait()

        def scatter(g, buf):
            dst = out_hbm.at[pl.ds(base + g * chunk, chunk)]
            pltpu.async_copy(rows_v.at[buf], dst, ssems[buf])

        def scatter_wait(buf):
            pltpu.make_async_copy(
                rows_v.at[buf], out_hbm.at[pl.ds(base, chunk)], ssems[buf]
            ).wait()

        for buf in range(NBUF):
            gather(buf, buf)

        def group(p, _):
            for buf in range(NBUF):
                gather_wait(buf)
                scatter(p * NBUF + buf, buf)

            @pl.when(p + 1 < ngroups)
            def _refill():
                for buf in range(NBUF):
                    scatter_wait(buf)
                    gather((p + 1) * NBUF + buf, buf)

            return ()

        lax.fori_loop(0, ngroups, group, ())
        for buf in range(NBUF):
            scatter_wait(buf)

    return body(idx, table)


def kernel(input_ids, table):
    ids_flat = input_ids.reshape(-1).astype(jnp.int32)
    b = ids_flat.shape[0]
    chunk = 80
    nchunk = (b // NW) // chunk
    assert b % (NW * chunk * NBUF) == 0
    out = _sc_gather(ids_flat, table, chunk=chunk, nchunk=nchunk)
    return out.reshape(input_ids.shape + (HIDDEN,))


# speedup vs baseline: 1.0199x; 1.0004x over previous
import functools

import jax
import jax.numpy as jnp
from jax import lax
from jax.experimental import pallas as pl
from jax.experimental.pallas import tpu as pltpu
from jax.experimental.pallas import tpu_sc as plsc

HIDDEN = 128
NUM_CORES = 2
NUM_SUBCORES = 16
NW = NUM_CORES * NUM_SUBCORES
NBUF = 10

_mesh = plsc.VectorSubcoreMesh(core_axis_name="c", subcore_axis_name="s")


@functools.partial(jax.jit, static_argnames=("chunk", "nchunk"))
def _sc_gather(idx, table, *, chunk, nchunk):
    b = idx.shape[0]
    bpw = b // NW
    ngroups = nchunk // NBUF

    @functools.partial(
        pl.kernel,
        mesh=_mesh,
        out_type=jax.ShapeDtypeStruct((b, HIDDEN), jnp.float32),
        scratch_types=[
            pltpu.VMEM((bpw,), jnp.int32),
            pltpu.VMEM((NBUF, chunk, HIDDEN), jnp.float32),
            pltpu.VMEM_SHARED((1000, HIDDEN), jnp.float32),
            [pltpu.SemaphoreType.DMA] * NBUF,
            [pltpu.SemaphoreType.DMA] * NBUF,
        ],
    )
    def body(idx_hbm, table_hbm, out_hbm, idx_v, rows_v, table_sh, gsems, ssems):
        sid = lax.axis_index("s")
        wid = sid * NUM_CORES + lax.axis_index("c")
        base = wid * bpw

        @pl.when(sid == 0)
        def _stage_table():
            pltpu.sync_copy(table_hbm, table_sh)

        pltpu.sync_copy(idx_hbm.at[pl.ds(base, bpw)], idx_v)
        plsc.subcore_barrier()

        def gather(g, buf):
            src = table_sh.at[idx_v.at[pl.ds(g * chunk, chunk)]]
            pltpu.async_copy(src, rows_v.at[buf], gsems[buf])

        def gather_wait(buf):
            pltpu.make_async_copy(
                table_hbm.at[pl.ds(0, chunk)], rows_v.at[buf], gsems[buf]
            ).wait()

        def scatter(g, buf):
            dst = out_hbm.at[pl.ds(base + g * chunk, chunk)]
            pltpu.async_copy(rows_v.at[buf], dst, ssems[buf])

        def scatter_wait(buf):
            pltpu.make_async_copy(
                rows_v.at[buf], out_hbm.at[pl.ds(base, chunk)], ssems[buf]
            ).wait()

        for buf in range(NBUF):
            gather(buf, buf)

        def group(p, _):
            for buf in range(NBUF):
                gather_wait(buf)
                scatter(p * NBUF + buf, buf)

            @pl.when(p + 1 < ngroups)
            def _refill():
                for buf in range(NBUF):
                    scatter_wait(buf)
                    gather((p + 1) * NBUF + buf, buf)

            return ()

        lax.fori_loop(0, ngroups, group, ())
        for buf in range(NBUF):
            scatter_wait(buf)

    return body(idx, table)


def kernel(input_ids, table):
    ids_flat = input_ids.reshape(-1).astype(jnp.int32)
    b = ids_flat.shape[0]
    chunk = 64
    nchunk = (b // NW) // chunk
    assert b % (NW * chunk * NBUF) == 0
    out = _sc_gather(ids_flat, table, chunk=chunk, nchunk=nchunk)
    return out.reshape(input_ids.shape + (HIDDEN,))
